# Initial kernel scaffold; baseline (speedup 1.0000x reference)
import jax
import jax.numpy as jnp
from jax.experimental import pallas as pl


def kernel(x, base_gate_w, base_up_w, base_down_w, router_weight, expert_gate_w, expert_up_w, expert_down_w):
    raise NotImplementedError("write your pallas kernel here")


# speedup vs baseline: 2.4574x; 2.4574x over previous
import functools

import jax
import jax.numpy as jnp
from jax.experimental import pallas as pl


BT = 256


def _moe_kernel(x_ref, wr_ref, wg_ref, wu_ref, wd_ref, out_ref, *, eff, n_exp):
    xb = x_ref[...]

    logits = jnp.dot(xb, wr_ref[...], preferred_element_type=jnp.float32)
    i1 = jnp.argmax(logits, axis=-1)
    v1 = jnp.max(logits, axis=-1)
    col = jax.lax.broadcasted_iota(jnp.int32, logits.shape, 1)
    masked = jnp.where(col == i1[:, None], -jnp.inf, logits)
    i2 = jnp.argmax(masked, axis=-1)
    v2 = jnp.max(masked, axis=-1)
    w1 = 1.0 / (1.0 + jnp.exp(v2 - v1))
    w2 = 1.0 - w1

    xb16 = xb.astype(jnp.bfloat16)
    g = jnp.dot(xb16, wg_ref[...], preferred_element_type=jnp.float32)
    u = jnp.dot(xb16, wu_ref[...], preferred_element_type=jnp.float32)
    h = (g * jax.lax.logistic(g)) * u

    e_lane = jax.lax.broadcasted_iota(jnp.int32, h.shape, 1) // eff
    gate = jnp.where(e_lane == i1[:, None], w1[:, None], 0.0) + jnp.where(
        e_lane == i2[:, None], w2[:, None], 0.0)
    hg = (h * gate).astype(jnp.bfloat16)

    out_ref[...] = jnp.dot(hg, wd_ref[...], preferred_element_type=jnp.float32)


def kernel(x, base_gate_w, base_up_w, base_down_w, router_weight,
           expert_gate_w, expert_up_w, expert_down_w):
    batch, seq_len, hidden = x.shape
    n_exp, eff, _ = expert_gate_w.shape
    t = batch * seq_len
    x2d = x.reshape(t, hidden)

    wr = router_weight.astype(jnp.float32).T
    wg = expert_gate_w.reshape(n_exp * eff, hidden).T.astype(jnp.bfloat16)
    wu = expert_up_w.reshape(n_exp * eff, hidden).T.astype(jnp.bfloat16)
    wd = expert_down_w.transpose(0, 2, 1).reshape(n_exp * eff, hidden).astype(jnp.bfloat16)

    grid = (t // BT,)
    out = pl.pallas_call(
        functools.partial(_moe_kernel, eff=eff, n_exp=n_exp),
        grid=grid,
        in_specs=[
            pl.BlockSpec((BT, hidden), lambda i: (i, 0)),
            pl.BlockSpec((hidden, n_exp), lambda i: (0, 0)),
            pl.BlockSpec((hidden, n_exp * eff), lambda i: (0, 0)),
            pl.BlockSpec((hidden, n_exp * eff), lambda i: (0, 0)),
            pl.BlockSpec((n_exp * eff, hidden), lambda i: (0, 0)),
        ],
        out_specs=pl.BlockSpec((BT, hidden), lambda i: (i, 0)),
        out_shape=jax.ShapeDtypeStruct((t, hidden), jnp.float32),
    )(x2d, wr, wg, wu, wd)

    return out.astype(x.dtype).reshape(batch, seq_len, hidden)


# speedup vs baseline: 2.5197x; 1.0253x over previous
import functools

import jax
import jax.numpy as jnp
from jax.experimental import pallas as pl


BT = 256


def _moe_kernel(x_ref, wr_ref, wg_ref, wu_ref, wd_ref, out_ref, *, eff, n_exp):
    xb = x_ref[...]

    logits = jax.lax.dot_general(xb, wr_ref[...], (((1,), (1,)), ((), ())),
                                 preferred_element_type=jnp.float32)
    i1 = jnp.argmax(logits, axis=-1)
    v1 = jnp.max(logits, axis=-1)
    col = jax.lax.broadcasted_iota(jnp.int32, logits.shape, 1)
    masked = jnp.where(col == i1[:, None], -jnp.inf, logits)
    i2 = jnp.argmax(masked, axis=-1)
    v2 = jnp.max(masked, axis=-1)
    w1 = 1.0 / (1.0 + jnp.exp(v2 - v1))
    w2 = 1.0 - w1

    xb16 = xb.astype(jnp.bfloat16)
    g = jax.lax.dot_general(xb16, wg_ref[...], (((1,), (1,)), ((), ())),
                            preferred_element_type=jnp.float32)
    u = jax.lax.dot_general(xb16, wu_ref[...], (((1,), (1,)), ((), ())),
                            preferred_element_type=jnp.float32)
    h = (g * jax.lax.logistic(g)) * u

    e_lane = jax.lax.broadcasted_iota(jnp.int32, h.shape, 1) // eff
    gate = jnp.where(e_lane == i1[:, None], w1[:, None], 0.0) + jnp.where(
        e_lane == i2[:, None], w2[:, None], 0.0)
    hg = (h * gate).astype(jnp.bfloat16)

    acc = jax.lax.dot_general(hg[:, 0:eff], wd_ref[0], (((1,), (1,)), ((), ())),
                              preferred_element_type=jnp.float32)
    for e in range(1, n_exp):
        acc = acc + jax.lax.dot_general(hg[:, e * eff:(e + 1) * eff], wd_ref[e],
                                        (((1,), (1,)), ((), ())),
                                        preferred_element_type=jnp.float32)
    out_ref[...] = acc


def kernel(x, base_gate_w, base_up_w, base_down_w, router_weight,
           expert_gate_w, expert_up_w, expert_down_w):
    batch, seq_len, hidden = x.shape
    n_exp, eff, _ = expert_gate_w.shape
    t = batch * seq_len
    x2d = x.reshape(t, hidden)

    wr = router_weight.astype(jnp.float32)
    wg = expert_gate_w.reshape(n_exp * eff, hidden).astype(jnp.bfloat16)
    wu = expert_up_w.reshape(n_exp * eff, hidden).astype(jnp.bfloat16)
    wd = expert_down_w.astype(jnp.bfloat16)

    grid = (t // BT,)
    out = pl.pallas_call(
        functools.partial(_moe_kernel, eff=eff, n_exp=n_exp),
        grid=grid,
        in_specs=[
            pl.BlockSpec((BT, hidden), lambda i: (i, 0)),
            pl.BlockSpec((n_exp, hidden), lambda i: (0, 0)),
            pl.BlockSpec((n_exp * eff, hidden), lambda i: (0, 0)),
            pl.BlockSpec((n_exp * eff, hidden), lambda i: (0, 0)),
            pl.BlockSpec((n_exp, hidden, eff), lambda i: (0, 0, 0)),
        ],
        out_specs=pl.BlockSpec((BT, hidden), lambda i: (i, 0)),
        out_shape=jax.ShapeDtypeStruct((t, hidden), jnp.float32),
    )(x2d, wr, wg, wu, wd)

    return out.astype(x.dtype).reshape(batch, seq_len, hidden)


# speedup vs baseline: 3.5350x; 1.4030x over previous
import functools

import jax
import jax.numpy as jnp
from jax.experimental import pallas as pl
from jax.experimental.pallas import tpu as pltpu


BT = 256


def _moe_kernel(x_ref, wr_ref, wg_ref, wu_ref, wd_ref, out_ref,
                wg16, wu16, wd16, *, eff, n_exp):
    @pl.when(pl.program_id(0) == 0)
    def _prep():
        wg16[...] = wg_ref[...].astype(jnp.bfloat16).T
        wu16[...] = wu_ref[...].astype(jnp.bfloat16).T
        for e in range(n_exp):
            wd16[e * eff:(e + 1) * eff, :] = wd_ref[e].astype(jnp.bfloat16).T

    xb = x_ref[...]

    logits = jax.lax.dot_general(xb, wr_ref[...], (((1,), (1,)), ((), ())),
                                 preferred_element_type=jnp.float32)
    i1 = jnp.argmax(logits, axis=-1)
    v1 = jnp.max(logits, axis=-1)
    col = jax.lax.broadcasted_iota(jnp.int32, logits.shape, 1)
    masked = jnp.where(col == i1[:, None], -jnp.inf, logits)
    i2 = jnp.argmax(masked, axis=-1)
    v2 = jnp.max(masked, axis=-1)
    w1 = 1.0 / (1.0 + jnp.exp(v2 - v1))
    w2 = 1.0 - w1

    xb16 = xb.astype(jnp.bfloat16)
    g = jnp.dot(xb16, wg16[...], preferred_element_type=jnp.float32)
    u = jnp.dot(xb16, wu16[...], preferred_element_type=jnp.float32)
    h = (g * jax.lax.logistic(g)) * u

    e_lane = jax.lax.broadcasted_iota(jnp.int32, h.shape, 1) // eff
    gate = jnp.where(e_lane == i1[:, None], w1[:, None], 0.0) + jnp.where(
        e_lane == i2[:, None], w2[:, None], 0.0)
    hg = (h * gate).astype(jnp.bfloat16)

    out_ref[...] = jnp.dot(hg, wd16[...], preferred_element_type=jnp.float32)


def kernel(x, base_gate_w, base_up_w, base_down_w, router_weight,
           expert_gate_w, expert_up_w, expert_down_w):
    batch, seq_len, hidden = x.shape
    n_exp, eff, _ = expert_gate_w.shape
    t = batch * seq_len
    x2d = x.reshape(t, hidden)

    wg = expert_gate_w.reshape(n_exp * eff, hidden)
    wu = expert_up_w.reshape(n_exp * eff, hidden)
    wd = expert_down_w

    grid = (t // BT,)
    out = pl.pallas_call(
        functools.partial(_moe_kernel, eff=eff, n_exp=n_exp),
        grid=grid,
        in_specs=[
            pl.BlockSpec((BT, hidden), lambda i: (i, 0)),
            pl.BlockSpec((n_exp, hidden), lambda i: (0, 0)),
            pl.BlockSpec((n_exp * eff, hidden), lambda i: (0, 0)),
            pl.BlockSpec((n_exp * eff, hidden), lambda i: (0, 0)),
            pl.BlockSpec((n_exp, hidden, eff), lambda i: (0, 0, 0)),
        ],
        out_specs=pl.BlockSpec((BT, hidden), lambda i: (i, 0)),
        out_shape=jax.ShapeDtypeStruct((t, hidden), jnp.float32),
        scratch_shapes=[
            pltpu.VMEM((hidden, n_exp * eff), jnp.bfloat16),
            pltpu.VMEM((hidden, n_exp * eff), jnp.bfloat16),
            pltpu.VMEM((n_exp * eff, hidden), jnp.bfloat16),
        ],
        compiler_params=pltpu.CompilerParams(
            vmem_limit_bytes=100 * 1024 * 1024,
        ),
    )(x2d, router_weight, wg, wu, wd)

    return out.astype(x.dtype).reshape(batch, seq_len, hidden)


# speedup vs baseline: 3.6137x; 1.0223x over previous
import functools

import jax
import jax.numpy as jnp
from jax.experimental import pallas as pl
from jax.experimental.pallas import tpu as pltpu


BT = 512


def _moe_kernel(x_ref, wr_ref, wg_ref, wu_ref, wd_ref, out_ref,
                wg16, wu16, wd16, *, eff, n_exp):
    @pl.when(pl.program_id(0) == 0)
    def _prep():
        wg16[...] = wg_ref[...].astype(jnp.bfloat16).T
        wu16[...] = wu_ref[...].astype(jnp.bfloat16).T
        for e in range(n_exp):
            wd16[e * eff:(e + 1) * eff, :] = wd_ref[e].astype(jnp.bfloat16).T

    xb = x_ref[...]

    logits = jax.lax.dot_general(xb, wr_ref[...], (((1,), (1,)), ((), ())),
                                 preferred_element_type=jnp.float32)
    i1 = jnp.argmax(logits, axis=-1)
    v1 = jnp.max(logits, axis=-1)
    col = jax.lax.broadcasted_iota(jnp.int32, logits.shape, 1)
    masked = jnp.where(col == i1[:, None], -jnp.inf, logits)
    i2 = jnp.argmax(masked, axis=-1)
    v2 = jnp.max(masked, axis=-1)
    w1 = 1.0 / (1.0 + jnp.exp(v2 - v1))
    w2 = 1.0 - w1

    xb16 = xb.astype(jnp.bfloat16)
    g = jnp.dot(xb16, wg16[...], preferred_element_type=jnp.float32)
    u = jnp.dot(xb16, wu16[...], preferred_element_type=jnp.float32)
    h = (g * jax.lax.logistic(g)) * u

    e_lane = jax.lax.broadcasted_iota(jnp.int32, h.shape, 1) // eff
    gate = jnp.where(e_lane == i1[:, None], w1[:, None], 0.0) + jnp.where(
        e_lane == i2[:, None], w2[:, None], 0.0)
    hg = (h * gate).astype(jnp.bfloat16)

    out_ref[...] = jnp.dot(hg, wd16[...], preferred_element_type=jnp.float32)


def kernel(x, base_gate_w, base_up_w, base_down_w, router_weight,
           expert_gate_w, expert_up_w, expert_down_w):
    batch, seq_len, hidden = x.shape
    n_exp, eff, _ = expert_gate_w.shape
    t = batch * seq_len
    x2d = x.reshape(t, hidden)

    wg = expert_gate_w.reshape(n_exp * eff, hidden)
    wu = expert_up_w.reshape(n_exp * eff, hidden)
    wd = expert_down_w

    grid = (t // BT,)
    out = pl.pallas_call(
        functools.partial(_moe_kernel, eff=eff, n_exp=n_exp),
        grid=grid,
        in_specs=[
            pl.BlockSpec((BT, hidden), lambda i: (i, 0)),
            pl.BlockSpec((n_exp, hidden), lambda i: (0, 0)),
            pl.BlockSpec((n_exp * eff, hidden), lambda i: (0, 0)),
            pl.BlockSpec((n_exp * eff, hidden), lambda i: (0, 0)),
            pl.BlockSpec((n_exp, hidden, eff), lambda i: (0, 0, 0)),
        ],
        out_specs=pl.BlockSpec((BT, hidden), lambda i: (i, 0)),
        out_shape=jax.ShapeDtypeStruct((t, hidden), jnp.float32),
        scratch_shapes=[
            pltpu.VMEM((hidden, n_exp * eff), jnp.bfloat16),
            pltpu.VMEM((hidden, n_exp * eff), jnp.bfloat16),
            pltpu.VMEM((n_exp * eff, hidden), jnp.bfloat16),
        ],
        compiler_params=pltpu.CompilerParams(
            vmem_limit_bytes=100 * 1024 * 1024,
        ),
    )(x2d, router_weight, wg, wu, wd)

    return out.astype(x.dtype).reshape(batch, seq_len, hidden)


# speedup vs baseline: 3.6347x; 1.0058x over previous
import functools

import jax
import jax.numpy as jnp
from jax.experimental import pallas as pl
from jax.experimental.pallas import tpu as pltpu


BT = 1024


def _moe_kernel(x_ref, wr_ref, wg_ref, wu_ref, wd_ref, out_ref,
                wg16, wu16, wd16, *, eff, n_exp):
    @pl.when(pl.program_id(0) == 0)
    def _prep():
        wg16[...] = wg_ref[...].astype(jnp.bfloat16).T
        wu16[...] = wu_ref[...].astype(jnp.bfloat16).T
        for e in range(n_exp):
            wd16[e * eff:(e + 1) * eff, :] = wd_ref[e].astype(jnp.bfloat16).T

    xb = x_ref[...]

    logits = jax.lax.dot_general(xb, wr_ref[...], (((1,), (1,)), ((), ())),
                                 preferred_element_type=jnp.float32)
    i1 = jnp.argmax(logits, axis=-1)
    v1 = jnp.max(logits, axis=-1)
    col = jax.lax.broadcasted_iota(jnp.int32, logits.shape, 1)
    masked = jnp.where(col == i1[:, None], -jnp.inf, logits)
    i2 = jnp.argmax(masked, axis=-1)
    v2 = jnp.max(masked, axis=-1)
    w1 = 1.0 / (1.0 + jnp.exp(v2 - v1))
    w2 = 1.0 - w1

    xb16 = xb.astype(jnp.bfloat16)
    g = jnp.dot(xb16, wg16[...], preferred_element_type=jnp.float32)
    u = jnp.dot(xb16, wu16[...], preferred_element_type=jnp.float32)
    h = (g * jax.lax.logistic(g)) * u

    e_lane = jax.lax.broadcasted_iota(jnp.int32, h.shape, 1) // eff
    gate = jnp.where(e_lane == i1[:, None], w1[:, None], 0.0) + jnp.where(
        e_lane == i2[:, None], w2[:, None], 0.0)
    hg = (h * gate).astype(jnp.bfloat16)

    out_ref[...] = jnp.dot(hg, wd16[...], preferred_element_type=jnp.float32)


def kernel(x, base_gate_w, base_up_w, base_down_w, router_weight,
           expert_gate_w, expert_up_w, expert_down_w):
    batch, seq_len, hidden = x.shape
    n_exp, eff, _ = expert_gate_w.shape
    t = batch * seq_len
    x2d = x.reshape(t, hidden)

    wg = expert_gate_w.reshape(n_exp * eff, hidden)
    wu = expert_up_w.reshape(n_exp * eff, hidden)
    wd = expert_down_w

    grid = (t // BT,)
    out = pl.pallas_call(
        functools.partial(_moe_kernel, eff=eff, n_exp=n_exp),
        grid=grid,
        in_specs=[
            pl.BlockSpec((BT, hidden), lambda i: (i, 0)),
            pl.BlockSpec((n_exp, hidden), lambda i: (0, 0)),
            pl.BlockSpec((n_exp * eff, hidden), lambda i: (0, 0)),
            pl.BlockSpec((n_exp * eff, hidden), lambda i: (0, 0)),
            pl.BlockSpec((n_exp, hidden, eff), lambda i: (0, 0, 0)),
        ],
        out_specs=pl.BlockSpec((BT, hidden), lambda i: (i, 0)),
        out_shape=jax.ShapeDtypeStruct((t, hidden), jnp.float32),
        scratch_shapes=[
            pltpu.VMEM((hidden, n_exp * eff), jnp.bfloat16),
            pltpu.VMEM((hidden, n_exp * eff), jnp.bfloat16),
            pltpu.VMEM((n_exp * eff, hidden), jnp.bfloat16),
        ],
        compiler_params=pltpu.CompilerParams(
            vmem_limit_bytes=100 * 1024 * 1024,
        ),
    )(x2d, router_weight, wg, wu, wd)

    return out.astype(x.dtype).reshape(batch, seq_len, hidden)
